# Initial kernel scaffold; baseline (speedup 1.0000x reference)
import functools

import jax
import jax.numpy as jnp
from jax import lax
from jax.experimental import pallas as pl
from jax.experimental.pallas import tpu as pltpu
from jax.experimental.pallas import tpu_sc as plsc

B = 16
K = 500
KP = 512
H = 152
W = 272
HW = H * W
NW = 32
HALF = KP // 2
NCH = HALF // 16
TBL = 1024


def _sc_body(ids_hbm, ids2_hbm, index2_hbm, wh2t_hbm, index_hbm, mask_hbm,
             wht_hbm, flow_hbm, pwh_hbm, out_hbm,
             ids_v, ids2_v, idsh_v, index2_v, wh2_v, idxh_v, maskh_v, whh_v,
             idxg_v, gath_v, t1_v, t2_v, outv, sem_a, sem_b):
    cid = lax.axis_index("c")
    sid = lax.axis_index("s")
    wid = sid * 2 + cid
    b = wid // 2
    half = wid % 2
    hbase = half * HALF

    cps = [
        pltpu.async_copy(ids_hbm.at[b], ids_v, sem_a),
        pltpu.async_copy(ids2_hbm.at[b], ids2_v, sem_a),
        pltpu.async_copy(ids_hbm.at[b, pl.ds(hbase, HALF)], idsh_v, sem_a),
        pltpu.async_copy(index2_hbm.at[b], index2_v, sem_a),
        pltpu.async_copy(wh2t_hbm.at[b], wh2_v, sem_a),
        pltpu.async_copy(index_hbm.at[b, pl.ds(hbase, HALF)], idxh_v, sem_a),
        pltpu.async_copy(mask_hbm.at[b, pl.ds(hbase, HALF)], maskh_v, sem_a),
        pltpu.async_copy(wht_hbm.at[b, :, pl.ds(hbase, HALF)], whh_v, sem_a),
    ]
    for cp in cps:
        cp.wait()

    for k in range(NCH):
        hw = idxh_v[pl.ds(k * 16, 16)]
        row = k // 8
        col = (k % 8) * 16
        for c in range(2):
            idxg_v[2 * c + row, pl.ds(col, 16)] = hw + (b * 2 + c) * HW
        for c in range(4):
            idxg_v[4 + 2 * c + row, pl.ds(col, 16)] = hw + (b * 4 + c) * HW

    gcs = [pltpu.async_copy(flow_hbm.at[idxg_v.at[r]], gath_v.at[r], sem_b)
           for r in range(4)]
    gcs += [pltpu.async_copy(pwh_hbm.at[idxg_v.at[r]], gath_v.at[r], sem_b)
            for r in range(4, 12)]

    big = jnp.full((16,), 1 << 20, jnp.int32)
    minus1 = jnp.full((16,), -1, jnp.int32)
    for j in range(TBL // 16):
        t1_v[pl.ds(j * 16, 16)] = big
        t2_v[pl.ds(j * 16, 16)] = minus1

    def build_t1(i, carry):
        x = (K - 1) - i
        t1_v[ids_v[x]] = x
        return carry

    lax.fori_loop(0, K, build_t1, 0)

    def build_t2(i, carry):
        t2_v[ids2_v[i]] = i
        return carry

    lax.fori_loop(0, K, build_t2, 0)

    for cp in gcs:
        cp.wait()

    iota = lax.iota(jnp.int32, 16)
    acc_xy = jnp.zeros((16,), jnp.float32)
    acc_wh = jnp.zeros((16,), jnp.float32)
    acc_m = jnp.zeros((16,), jnp.float32)
    for k in range(NCH):
        s = k * 16
        row = k // 8
        col = (k % 8) * 16
        v = idsh_v[pl.ds(s, 16)]
        t1 = plsc.load_gather(t1_v, [v])
        t2 = plsc.load_gather(t2_v, [v])
        xg = iota + (s + hbase)
        recv = (v != 0) & (t1 == xg) & (t2 >= 0) & (xg < K)
        w = jnp.maximum(t2, 0)
        ix2 = plsc.load_gather(index2_v, [w])
        xc = (ix2 % W).astype(jnp.float32)
        yc = ix2.astype(jnp.float32) / float(W)
        rin0 = jnp.where(recv, xc, 0.0)
        rin1 = jnp.where(recv, yc, 0.0)
        m = maskh_v[pl.ds(s, 16)].astype(jnp.float32)
        f0 = gath_v[row, pl.ds(col, 16)]
        f1 = gath_v[2 + row, pl.ds(col, 16)]
        acc_xy = acc_xy + jnp.abs(f0 * m - rin0) + jnp.abs(f1 * m - rin1)
        for c in range(4):
            rw = jnp.where(
                recv,
                plsc.load_gather(wh2_v, [jnp.full((16,), c, jnp.int32), w]),
                0.0)
            whc = whh_v[c, pl.ds(s, 16)]
            d = rw - whc
            term = jnp.where(d != -whc, d, 0.0) * m
            pw = gath_v[4 + 2 * c + row, pl.ds(col, 16)]
            acc_wh = acc_wh + jnp.abs(pw * m - term)
        acc_m = acc_m + m

    outv[0, :] = acc_xy
    outv[1, :] = acc_wh
    outv[2, :] = acc_m
    pltpu.sync_copy(outv, out_hbm.at[wid])


_sc_call = functools.partial(
    pl.kernel,
    out_type=jax.ShapeDtypeStruct((NW, 3, 16), jnp.float32),
    mesh=plsc.VectorSubcoreMesh(core_axis_name="c", subcore_axis_name="s"),
    scratch_types=[
        pltpu.VMEM((KP,), jnp.int32),
        pltpu.VMEM((KP,), jnp.int32),
        pltpu.VMEM((HALF,), jnp.int32),
        pltpu.VMEM((KP,), jnp.int32),
        pltpu.VMEM((4, KP), jnp.float32),
        pltpu.VMEM((HALF,), jnp.int32),
        pltpu.VMEM((HALF,), jnp.int32),
        pltpu.VMEM((4, HALF), jnp.float32),
        pltpu.VMEM((12, 128), jnp.int32),
        pltpu.VMEM((12, 128), jnp.float32),
        pltpu.VMEM((TBL,), jnp.int32),
        pltpu.VMEM((TBL,), jnp.int32),
        pltpu.VMEM((3, 16), jnp.float32),
        pltpu.SemaphoreType.DMA,
        pltpu.SemaphoreType.DMA,
    ],
)


def kernel(flow, p_wh, mask, index, ids, wh, index2, ids2, wh2):
    pads = ((0, 0), (0, KP - K))
    ids_p = jnp.pad(ids, pads)
    ids2_p = jnp.pad(ids2, pads)
    index2_p = jnp.pad(index2, pads)
    index_p = jnp.pad(index, pads)
    mask_p = jnp.pad(mask, pads)
    pads3 = ((0, 0), (0, 0), (0, KP - K))
    wht = jnp.pad(wh.transpose(0, 2, 1), pads3)
    wh2t = jnp.pad(wh2.transpose(0, 2, 1), pads3)
    flow_flat = flow.reshape(-1)
    pwh_flat = p_wh.reshape(-1)
    parts = _sc_call(_sc_body)(ids_p, ids2_p, index2_p, wh2t, index_p, mask_p,
                               wht, flow_flat, pwh_flat)
    s = parts.sum(axis=(0, 2))
    loss = s[0] / (2.0 * s[2] + 1e-4)
    wh_loss = s[1] / (4.0 * s[2] + 1e-4)
    return (loss, wh_loss)


# speedup vs baseline: 2.7418x; 2.7418x over previous
import functools

import jax
import jax.numpy as jnp
from jax import lax
from jax.experimental import pallas as pl
from jax.experimental.pallas import tpu as pltpu
from jax.experimental.pallas import tpu_sc as plsc

B = 16
K = 500
KP = 512
H = 152
W = 272
HW = H * W
NW = 32
HALF = KP // 2
NCH = HALF // 16
TBL = 1024


def _sc_body(ids_hbm, ids2_hbm, index2_hbm, wh2t_hbm, index_hbm, mask_hbm,
             wht_hbm, flow_hbm, pwh_hbm, out_hbm,
             ids_v, ids2_v, idsh_v, index2_v, wh2_v, idxh_v, maskh_v, whh_v,
             idxg_v, gath_v, t1_v, t2_v, outv, sem_a, sem_b):
    cid = lax.axis_index("c")
    sid = lax.axis_index("s")
    wid = sid * 2 + cid
    b = wid // 2
    half = wid % 2
    hbase = half * HALF

    cps = [
        pltpu.async_copy(ids_hbm.at[b], ids_v, sem_a),
        pltpu.async_copy(ids2_hbm.at[b], ids2_v, sem_a),
        pltpu.async_copy(ids_hbm.at[b, pl.ds(hbase, HALF)], idsh_v, sem_a),
        pltpu.async_copy(index2_hbm.at[b], index2_v, sem_a),
        pltpu.async_copy(wh2t_hbm.at[b], wh2_v, sem_a),
        pltpu.async_copy(index_hbm.at[b, pl.ds(hbase, HALF)], idxh_v, sem_a),
        pltpu.async_copy(mask_hbm.at[b, pl.ds(hbase, HALF)], maskh_v, sem_a),
        pltpu.async_copy(wht_hbm.at[b, :, pl.ds(hbase, HALF)], whh_v, sem_a),
    ]
    for cp in cps:
        cp.wait()

    for k in range(NCH):
        hw = idxh_v[pl.ds(k * 16, 16)]
        row = k // 8
        col = (k % 8) * 16
        for c in range(2):
            idxg_v[2 * c + row, pl.ds(col, 16)] = hw + (b * 2 + c) * HW
        for c in range(4):
            idxg_v[4 + 2 * c + row, pl.ds(col, 16)] = hw + (b * 4 + c) * HW

    gcs = [pltpu.async_copy(flow_hbm.at[idxg_v.at[r]], gath_v.at[r], sem_b)
           for r in range(4)]
    gcs += [pltpu.async_copy(pwh_hbm.at[idxg_v.at[r]], gath_v.at[r], sem_b)
            for r in range(4, 12)]

    big = jnp.full((16,), 1 << 20, jnp.int32)
    minus1 = jnp.full((16,), -1, jnp.int32)
    for j in range(TBL // 16):
        t1_v[pl.ds(j * 16, 16)] = big
        t2_v[pl.ds(j * 16, 16)] = minus1

    def shift_lanes(x, idx):
        dnums = lax.GatherDimensionNumbers(
            offset_dims=(), collapsed_slice_dims=(0,), start_index_map=(0,))
        return lax.gather(x, idx[:, None], dnums, (1,),
                          mode=lax.GatherScatterMode.PROMISE_IN_BOUNDS)

    iota = lax.iota(jnp.int32, 16)
    shift_up = jnp.minimum(iota + 1, 15)
    shift_dn = jnp.maximum(iota - 1, 0)
    for k in range(KP // 16 - 1, -1, -1):
        xg = iota + k * 16
        v = ids_v[pl.ds(k * 16, 16)]
        ks, xs = plsc.sort_key_val(v * 16 + iota, xg)
        vs = lax.shift_right_logical(ks, 4)
        vprev = shift_lanes(vs, shift_dn)
        winner = (vs != vprev) | (iota == 0)
        plsc.store_scatter(t1_v, [vs], xs, mask=winner)
    for k in range(KP // 16):
        ig = iota + k * 16
        v = ids2_v[pl.ds(k * 16, 16)]
        ks, isrt = plsc.sort_key_val(v * 16 + iota, ig)
        vs = lax.shift_right_logical(ks, 4)
        vnext = shift_lanes(vs, shift_up)
        winner = (vnext != vs) | (iota == 15)
        plsc.store_scatter(t2_v, [vs], isrt, mask=winner)

    for cp in gcs:
        cp.wait()

    acc_xy = jnp.zeros((16,), jnp.float32)
    acc_wh = jnp.zeros((16,), jnp.float32)
    acc_m = jnp.zeros((16,), jnp.float32)
    for k in range(NCH):
        s = k * 16
        row = k // 8
        col = (k % 8) * 16
        v = idsh_v[pl.ds(s, 16)]
        t1 = plsc.load_gather(t1_v, [v])
        t2 = plsc.load_gather(t2_v, [v])
        xg = iota + (s + hbase)
        recv = (v != 0) & (t1 == xg) & (t2 >= 0) & (xg < K)
        w = jnp.maximum(t2, 0)
        ix2 = plsc.load_gather(index2_v, [w])
        xc = (ix2 % W).astype(jnp.float32)
        yc = ix2.astype(jnp.float32) / float(W)
        rin0 = jnp.where(recv, xc, 0.0)
        rin1 = jnp.where(recv, yc, 0.0)
        m = maskh_v[pl.ds(s, 16)].astype(jnp.float32)
        f0 = gath_v[row, pl.ds(col, 16)]
        f1 = gath_v[2 + row, pl.ds(col, 16)]
        acc_xy = acc_xy + jnp.abs(f0 * m - rin0) + jnp.abs(f1 * m - rin1)
        for c in range(4):
            rw = jnp.where(
                recv,
                plsc.load_gather(wh2_v, [jnp.full((16,), c, jnp.int32), w]),
                0.0)
            whc = whh_v[c, pl.ds(s, 16)]
            d = rw - whc
            term = jnp.where(d != -whc, d, 0.0) * m
            pw = gath_v[4 + 2 * c + row, pl.ds(col, 16)]
            acc_wh = acc_wh + jnp.abs(pw * m - term)
        acc_m = acc_m + m

    outv[0, :] = acc_xy
    outv[1, :] = acc_wh
    outv[2, :] = acc_m
    pltpu.sync_copy(outv, out_hbm.at[wid])


_sc_call = functools.partial(
    pl.kernel,
    out_type=jax.ShapeDtypeStruct((NW, 3, 16), jnp.float32),
    mesh=plsc.VectorSubcoreMesh(core_axis_name="c", subcore_axis_name="s"),
    compiler_params=pltpu.CompilerParams(needs_layout_passes=False),
    scratch_types=[
        pltpu.VMEM((KP,), jnp.int32),
        pltpu.VMEM((KP,), jnp.int32),
        pltpu.VMEM((HALF,), jnp.int32),
        pltpu.VMEM((KP,), jnp.int32),
        pltpu.VMEM((4, KP), jnp.float32),
        pltpu.VMEM((HALF,), jnp.int32),
        pltpu.VMEM((HALF,), jnp.int32),
        pltpu.VMEM((4, HALF), jnp.float32),
        pltpu.VMEM((12, 128), jnp.int32),
        pltpu.VMEM((12, 128), jnp.float32),
        pltpu.VMEM((TBL,), jnp.int32),
        pltpu.VMEM((TBL,), jnp.int32),
        pltpu.VMEM((3, 16), jnp.float32),
        pltpu.SemaphoreType.DMA,
        pltpu.SemaphoreType.DMA,
    ],
)


def kernel(flow, p_wh, mask, index, ids, wh, index2, ids2, wh2):
    pads = ((0, 0), (0, KP - K))
    ids_p = jnp.pad(ids, pads)
    ids2_p = jnp.pad(ids2, pads)
    index2_p = jnp.pad(index2, pads)
    index_p = jnp.pad(index, pads)
    mask_p = jnp.pad(mask, pads)
    pads3 = ((0, 0), (0, 0), (0, KP - K))
    wht = jnp.pad(wh.transpose(0, 2, 1), pads3)
    wh2t = jnp.pad(wh2.transpose(0, 2, 1), pads3)
    flow_flat = flow.reshape(-1)
    pwh_flat = p_wh.reshape(-1)
    parts = _sc_call(_sc_body)(ids_p, ids2_p, index2_p, wh2t, index_p, mask_p,
                               wht, flow_flat, pwh_flat)
    s = parts.sum(axis=(0, 2))
    loss = s[0] / (2.0 * s[2] + 1e-4)
    wh_loss = s[1] / (4.0 * s[2] + 1e-4)
    return (loss, wh_loss)


# speedup vs baseline: 2.8989x; 1.0573x over previous
import functools

import jax
import jax.numpy as jnp
from jax import lax
from jax.experimental import pallas as pl
from jax.experimental.pallas import tpu as pltpu
from jax.experimental.pallas import tpu_sc as plsc

B = 16
K = 500
KP = 512
H = 152
W = 272
HW = H * W
NW = 32
HALF = KP // 2
NCH = HALF // 16
TBL = 1024


def _sc_body(ints_hbm, flts_hbm, flow_hbm, pwh_hbm, out_hbm,
             ids_v, ids2_v, idsh_v, index2_v, wh2_v, idxh_v, maskh_v, whh_v,
             idxg_v, gath_v, t1_v, t2_v, outv, sem_a, sem_b):
    cid = lax.axis_index("c")
    sid = lax.axis_index("s")
    wid = sid * 2 + cid
    b = wid // 2
    half = wid % 2
    hbase = half * HALF

    ib = b * 5
    fb = b * 2
    cps = [
        pltpu.async_copy(ints_hbm.at[ib + 0], ids_v, sem_a),
        pltpu.async_copy(ints_hbm.at[ib + 1], ids2_v, sem_a),
        pltpu.async_copy(ints_hbm.at[ib + 0, pl.ds(hbase, HALF)], idsh_v, sem_a),
        pltpu.async_copy(ints_hbm.at[ib + 2], index2_v, sem_a),
        pltpu.async_copy(flts_hbm.at[fb + 1], wh2_v, sem_a),
        pltpu.async_copy(ints_hbm.at[ib + 3, pl.ds(hbase, HALF)], idxh_v, sem_a),
        pltpu.async_copy(ints_hbm.at[ib + 4, pl.ds(hbase, HALF)], maskh_v, sem_a),
        pltpu.async_copy(flts_hbm.at[fb + 0, :, pl.ds(hbase, HALF)], whh_v, sem_a),
    ]
    for cp in cps:
        cp.wait()

    for k in range(NCH):
        hw = idxh_v[pl.ds(k * 16, 16)]
        row = k // 8
        col = (k % 8) * 16
        for c in range(2):
            idxg_v[2 * c + row, pl.ds(col, 16)] = hw + (b * 2 + c) * HW
        for c in range(4):
            idxg_v[4 + 2 * c + row, pl.ds(col, 16)] = hw + (b * 4 + c) * HW

    gcs = [pltpu.async_copy(flow_hbm.at[idxg_v.at[r]], gath_v.at[r], sem_b)
           for r in range(4)]
    gcs += [pltpu.async_copy(pwh_hbm.at[idxg_v.at[r]], gath_v.at[r], sem_b)
            for r in range(4, 12)]

    big = jnp.full((16,), 1 << 20, jnp.int32)
    minus1 = jnp.full((16,), -1, jnp.int32)
    for j in range(TBL // 16):
        t1_v[pl.ds(j * 16, 16)] = big
        t2_v[pl.ds(j * 16, 16)] = minus1

    def shift_lanes(x, idx):
        dnums = lax.GatherDimensionNumbers(
            offset_dims=(), collapsed_slice_dims=(0,), start_index_map=(0,))
        return lax.gather(x, idx[:, None], dnums, (1,),
                          mode=lax.GatherScatterMode.PROMISE_IN_BOUNDS)

    iota = lax.iota(jnp.int32, 16)
    shift_up = jnp.minimum(iota + 1, 15)
    shift_dn = jnp.maximum(iota - 1, 0)
    for k in range(KP // 16 - 1, -1, -1):
        xg = iota + k * 16
        v = ids_v[pl.ds(k * 16, 16)]
        ks, xs = plsc.sort_key_val(v * 16 + iota, xg)
        vs = lax.shift_right_logical(ks, 4)
        vprev = shift_lanes(vs, shift_dn)
        winner = (vs != vprev) | (iota == 0)
        plsc.store_scatter(t1_v, [vs], xs, mask=winner)
    for k in range(KP // 16):
        ig = iota + k * 16
        v = ids2_v[pl.ds(k * 16, 16)]
        ks, isrt = plsc.sort_key_val(v * 16 + iota, ig)
        vs = lax.shift_right_logical(ks, 4)
        vnext = shift_lanes(vs, shift_up)
        winner = (vnext != vs) | (iota == 15)
        plsc.store_scatter(t2_v, [vs], isrt, mask=winner)

    for cp in gcs:
        cp.wait()

    acc_xy = jnp.zeros((16,), jnp.float32)
    acc_wh = jnp.zeros((16,), jnp.float32)
    acc_m = jnp.zeros((16,), jnp.float32)
    for k in range(NCH):
        s = k * 16
        row = k // 8
        col = (k % 8) * 16
        v = idsh_v[pl.ds(s, 16)]
        t1 = plsc.load_gather(t1_v, [v])
        t2 = plsc.load_gather(t2_v, [v])
        xg = iota + (s + hbase)
        recv = (v != 0) & (t1 == xg) & (t2 >= 0) & (xg < K)
        w = jnp.maximum(t2, 0)
        ix2 = plsc.load_gather(index2_v, [w])
        xc = (ix2 % W).astype(jnp.float32)
        yc = ix2.astype(jnp.float32) / float(W)
        rin0 = jnp.where(recv, xc, 0.0)
        rin1 = jnp.where(recv, yc, 0.0)
        m = maskh_v[pl.ds(s, 16)].astype(jnp.float32)
        f0 = gath_v[row, pl.ds(col, 16)]
        f1 = gath_v[2 + row, pl.ds(col, 16)]
        acc_xy = acc_xy + jnp.abs(f0 * m - rin0) + jnp.abs(f1 * m - rin1)
        for c in range(4):
            rw = jnp.where(
                recv,
                plsc.load_gather(wh2_v, [jnp.full((16,), c, jnp.int32), w]),
                0.0)
            whc = whh_v[c, pl.ds(s, 16)]
            d = rw - whc
            term = jnp.where(d != -whc, d, 0.0) * m
            pw = gath_v[4 + 2 * c + row, pl.ds(col, 16)]
            acc_wh = acc_wh + jnp.abs(pw * m - term)
        acc_m = acc_m + m

    outv[0, :] = acc_xy
    outv[1, :] = acc_wh
    outv[2, :] = acc_m
    pltpu.sync_copy(outv, out_hbm.at[wid])


_sc_call = functools.partial(
    pl.kernel,
    out_type=jax.ShapeDtypeStruct((NW, 3, 16), jnp.float32),
    mesh=plsc.VectorSubcoreMesh(core_axis_name="c", subcore_axis_name="s"),
    compiler_params=pltpu.CompilerParams(needs_layout_passes=False),
    scratch_types=[
        pltpu.VMEM((KP,), jnp.int32),
        pltpu.VMEM((KP,), jnp.int32),
        pltpu.VMEM((HALF,), jnp.int32),
        pltpu.VMEM((KP,), jnp.int32),
        pltpu.VMEM((4, KP), jnp.float32),
        pltpu.VMEM((HALF,), jnp.int32),
        pltpu.VMEM((HALF,), jnp.int32),
        pltpu.VMEM((4, HALF), jnp.float32),
        pltpu.VMEM((12, 128), jnp.int32),
        pltpu.VMEM((12, 128), jnp.float32),
        pltpu.VMEM((TBL,), jnp.int32),
        pltpu.VMEM((TBL,), jnp.int32),
        pltpu.VMEM((3, 16), jnp.float32),
        pltpu.SemaphoreType.DMA,
        pltpu.SemaphoreType.DMA,
    ],
)


def kernel(flow, p_wh, mask, index, ids, wh, index2, ids2, wh2):
    ints = jnp.pad(jnp.stack([ids, ids2, index2, index, mask], axis=1),
                   ((0, 0), (0, 0), (0, KP - K))).reshape(B * 5, KP)
    flts = jnp.pad(jnp.stack([wh, wh2], axis=1).transpose(0, 1, 3, 2),
                   ((0, 0), (0, 0), (0, 0), (0, KP - K))).reshape(B * 2, 4, KP)
    flow_flat = flow.reshape(-1)
    pwh_flat = p_wh.reshape(-1)
    parts = _sc_call(_sc_body)(ints, flts, flow_flat, pwh_flat)
    s = parts.sum(axis=(0, 2))
    loss = s[0] / (2.0 * s[2] + 1e-4)
    wh_loss = s[1] / (4.0 * s[2] + 1e-4)
    return (loss, wh_loss)
